# Initial kernel scaffold; baseline (speedup 1.0000x reference)
import jax
import jax.numpy as jnp
from jax.experimental import pallas as pl


def kernel(anchor, positive, negative, table):
    raise NotImplementedError("write your pallas kernel here")


# speedup vs baseline: 2.2979x; 2.2979x over previous
import functools

import jax
import jax.numpy as jnp
from jax import lax
from jax.experimental import pallas as pl
from jax.experimental.pallas import tpu as pltpu
from jax.experimental.pallas import tpu_sc as plsc

NUM_EMB = 1000
EMB_DIM = 128
BATCH = 16384
LANES = 16
VREGS_PER_ROW = EMB_DIM // LANES
EPS = 1e-6
MARGIN = 1.0

_info = plsc.get_sparse_core_info()
_NC, _NS = _info.num_cores, _info.num_subcores
NW = _NC * _NS
B_PER_W = BATCH // NW
CHUNK = 64
N_CHUNKS = B_PER_W // CHUNK


def _sc_body(table_hbm, a_hbm, p_hbm, n_hbm, sq_ap_hbm, sq_an_hbm,
             idx_a, idx_p, idx_n, rows_a, rows_p, rows_n,
             sq_ap_v, sq_an_v, sem):
    wid = lax.axis_index("s") * _NC + lax.axis_index("c")
    base = wid * B_PER_W

    for c in range(N_CHUNKS):
        off = base + c * CHUNK
        pltpu.sync_copy(a_hbm.at[pl.ds(off, CHUNK)], idx_a)
        pltpu.sync_copy(p_hbm.at[pl.ds(off, CHUNK)], idx_p)
        pltpu.sync_copy(n_hbm.at[pl.ds(off, CHUNK)], idx_n)
        ca = pltpu.async_copy(table_hbm.at[idx_a], rows_a, sem)
        cp = pltpu.async_copy(table_hbm.at[idx_p], rows_p, sem)
        cn = pltpu.async_copy(table_hbm.at[idx_n], rows_n, sem)
        ca.wait()
        cp.wait()
        cn.wait()

        def row_body(i, carry, c=c):
            acc_ap = jnp.zeros((LANES,), jnp.float32)
            acc_an = jnp.zeros((LANES,), jnp.float32)
            for v in range(VREGS_PER_ROW):
                a = rows_a[i, pl.ds(v * LANES, LANES)] + EPS
                d_ap = a - rows_p[i, pl.ds(v * LANES, LANES)]
                d_an = a - rows_n[i, pl.ds(v * LANES, LANES)]
                acc_ap = acc_ap + d_ap * d_ap
                acc_an = acc_an + d_an * d_an
            sq_ap_v[c * CHUNK + i, :] = acc_ap
            sq_an_v[c * CHUNK + i, :] = acc_an
            return carry

        lax.fori_loop(0, CHUNK, row_body, 0)

    pltpu.sync_copy(sq_ap_v, sq_ap_hbm.at[pl.ds(base, B_PER_W)])
    pltpu.sync_copy(sq_an_v, sq_an_hbm.at[pl.ds(base, B_PER_W)])


_sc_gather_dist = pl.kernel(
    _sc_body,
    mesh=plsc.VectorSubcoreMesh(core_axis_name="c", subcore_axis_name="s"),
    compiler_params=pltpu.CompilerParams(use_tc_tiling_on_sc=False),
    out_type=[
        jax.ShapeDtypeStruct((BATCH, LANES), jnp.float32),
        jax.ShapeDtypeStruct((BATCH, LANES), jnp.float32),
    ],
    scratch_types=[
        pltpu.VMEM((CHUNK,), jnp.int32),
        pltpu.VMEM((CHUNK,), jnp.int32),
        pltpu.VMEM((CHUNK,), jnp.int32),
        pltpu.VMEM((CHUNK, EMB_DIM), jnp.float32),
        pltpu.VMEM((CHUNK, EMB_DIM), jnp.float32),
        pltpu.VMEM((CHUNK, EMB_DIM), jnp.float32),
        pltpu.VMEM((B_PER_W, LANES), jnp.float32),
        pltpu.VMEM((B_PER_W, LANES), jnp.float32),
        pltpu.SemaphoreType.DMA,
    ],
)


def _tc_loss_body(sq_ap_ref, sq_an_ref, out_ref):
    d_ap = jnp.sqrt(jnp.sum(sq_ap_ref[...], axis=-1))
    d_an = jnp.sqrt(jnp.sum(sq_an_ref[...], axis=-1))
    t = jnp.maximum(d_ap - d_an + MARGIN, 0.0)
    out_ref[0, 0] = jnp.sum(t) / BATCH


def kernel(anchor, positive, negative, table):
    sq_ap, sq_an = _sc_gather_dist(
        table, anchor.astype(jnp.int32), positive.astype(jnp.int32),
        negative.astype(jnp.int32))
    loss = pl.pallas_call(
        _tc_loss_body,
        out_shape=jax.ShapeDtypeStruct((1, 1), jnp.float32),
        in_specs=[
            pl.BlockSpec(memory_space=pltpu.VMEM),
            pl.BlockSpec(memory_space=pltpu.VMEM),
        ],
        out_specs=pl.BlockSpec(memory_space=pltpu.SMEM),
    )(sq_ap, sq_an)
    return loss.reshape(())


# speedup vs baseline: 4.0597x; 1.7667x over previous
import jax
import jax.numpy as jnp
from jax import lax
from jax.experimental import pallas as pl
from jax.experimental.pallas import tpu as pltpu
from jax.experimental.pallas import tpu_sc as plsc

NUM_EMB = 1000
EMB_DIM = 128
BATCH = 16384
LANES = 16
VREGS_PER_ROW = EMB_DIM // LANES
EPS = 1e-6
MARGIN = 1.0

_info = plsc.get_sparse_core_info()
_NC, _NS = _info.num_cores, _info.num_subcores
NW = _NC * _NS
B_PER_W = BATCH // NW
CHUNK = 64
N_CHUNKS = B_PER_W // CHUNK
OUT_ROWS = BATCH // 8
OUT_ROWS_W = B_PER_W // 8


def _sc_body(table_hbm, a_hbm, p_hbm, n_hbm, sq_ap_hbm, sq_an_hbm,
             idx_a, idx_p, idx_n,
             ra0, rp0, rn0, ra1, rp1, rn1,
             sq_ap_v, sq_an_v, sem0, sem1):
    wid = lax.axis_index("s") * _NC + lax.axis_index("c")
    base = wid * B_PER_W

    pltpu.sync_copy(a_hbm.at[pl.ds(base, B_PER_W)], idx_a)
    pltpu.sync_copy(p_hbm.at[pl.ds(base, B_PER_W)], idx_p)
    pltpu.sync_copy(n_hbm.at[pl.ds(base, B_PER_W)], idx_n)

    bufs = ((ra0, rp0, rn0, sem0), (ra1, rp1, rn1, sem1))

    def issue(c):
        ba, bp, bn, sem = bufs[c & 1]
        s = pl.ds(c * CHUNK, CHUNK)
        return (pltpu.async_copy(table_hbm.at[idx_a.at[s]], ba, sem),
                pltpu.async_copy(table_hbm.at[idx_p.at[s]], bp, sem),
                pltpu.async_copy(table_hbm.at[idx_n.at[s]], bn, sem))

    inflight = issue(0)
    for c in range(N_CHUNKS):
        ba, bp, bn, _ = bufs[c & 1]
        nxt = issue(c + 1) if c + 1 < N_CHUNKS else None
        for d in inflight:
            d.wait()
        inflight = nxt

        def row_body(i, carry, c=c, ba=ba, bp=bp, bn=bn):
            acc_ap = jnp.zeros((LANES,), jnp.float32)
            acc_an = jnp.zeros((LANES,), jnp.float32)
            for v in range(VREGS_PER_ROW):
                a = ba[i, pl.ds(v * LANES, LANES)] + EPS
                d_ap = a - bp[i, pl.ds(v * LANES, LANES)]
                d_an = a - bn[i, pl.ds(v * LANES, LANES)]
                acc_ap = acc_ap + d_ap * d_ap
                acc_an = acc_an + d_an * d_an
            j = c * CHUNK + i
            row = j >> 3
            lane = (j & 7) * LANES
            sq_ap_v[row, pl.ds(lane, LANES)] = acc_ap
            sq_an_v[row, pl.ds(lane, LANES)] = acc_an
            return carry

        lax.fori_loop(0, CHUNK, row_body, 0)

    pltpu.sync_copy(sq_ap_v, sq_ap_hbm.at[pl.ds(wid * OUT_ROWS_W, OUT_ROWS_W)])
    pltpu.sync_copy(sq_an_v, sq_an_hbm.at[pl.ds(wid * OUT_ROWS_W, OUT_ROWS_W)])


_sc_gather_dist = pl.kernel(
    _sc_body,
    mesh=plsc.VectorSubcoreMesh(core_axis_name="c", subcore_axis_name="s"),
    compiler_params=pltpu.CompilerParams(use_tc_tiling_on_sc=False),
    out_type=[
        jax.ShapeDtypeStruct((OUT_ROWS, EMB_DIM), jnp.float32),
        jax.ShapeDtypeStruct((OUT_ROWS, EMB_DIM), jnp.float32),
    ],
    scratch_types=[
        pltpu.VMEM((B_PER_W,), jnp.int32),
        pltpu.VMEM((B_PER_W,), jnp.int32),
        pltpu.VMEM((B_PER_W,), jnp.int32),
        pltpu.VMEM((CHUNK, EMB_DIM), jnp.float32),
        pltpu.VMEM((CHUNK, EMB_DIM), jnp.float32),
        pltpu.VMEM((CHUNK, EMB_DIM), jnp.float32),
        pltpu.VMEM((CHUNK, EMB_DIM), jnp.float32),
        pltpu.VMEM((CHUNK, EMB_DIM), jnp.float32),
        pltpu.VMEM((CHUNK, EMB_DIM), jnp.float32),
        pltpu.VMEM((OUT_ROWS_W, EMB_DIM), jnp.float32),
        pltpu.VMEM((OUT_ROWS_W, EMB_DIM), jnp.float32),
        pltpu.SemaphoreType.DMA,
        pltpu.SemaphoreType.DMA,
    ],
)


def _tc_loss_body(sq_ap_ref, sq_an_ref, out_ref):
    k = lax.broadcasted_iota(jnp.int32, (EMB_DIM, 8), 0) // LANES
    s = lax.broadcasted_iota(jnp.int32, (EMB_DIM, 8), 1)
    seg = (k == s).astype(jnp.float32)
    d2_ap = jnp.dot(sq_ap_ref[...], seg, preferred_element_type=jnp.float32)
    d2_an = jnp.dot(sq_an_ref[...], seg, preferred_element_type=jnp.float32)
    t = jnp.maximum(jnp.sqrt(d2_ap) - jnp.sqrt(d2_an) + MARGIN, 0.0)
    out_ref[0, 0] = jnp.sum(t) / BATCH


def kernel(anchor, positive, negative, table):
    sq_ap, sq_an = _sc_gather_dist(
        table, anchor.astype(jnp.int32), positive.astype(jnp.int32),
        negative.astype(jnp.int32))
    loss = pl.pallas_call(
        _tc_loss_body,
        out_shape=jax.ShapeDtypeStruct((1, 1), jnp.float32),
        in_specs=[
            pl.BlockSpec(memory_space=pltpu.VMEM),
            pl.BlockSpec(memory_space=pltpu.VMEM),
        ],
        out_specs=pl.BlockSpec(memory_space=pltpu.SMEM),
    )(sq_ap, sq_an)
    return loss.reshape(())
